# baseline (speedup 1.0000x reference)
import functools

import jax
import jax.numpy as jnp
from jax import lax
from jax.experimental import pallas as pl
from jax.experimental.pallas import tpu as pltpu
from jax.experimental.pallas import tpu_sc as plsc

NC = 2
NS = 16
LANES = 128
SB = 16
SBE = 8
D = 16


def _mesh():
    return plsc.VectorSubcoreMesh(core_axis_name="c", subcore_axis_name="s")


_SC_PARAMS = pltpu.CompilerParams(use_tc_tiling_on_sc=False)


def _sc_degree(dst_flat, n_pad):
    ep = dst_flat.shape[0]
    ew = ep // (NC * NS)
    ch = SB * LANES
    outer = ew // ch
    rps = n_pad // NS

    @functools.partial(
        pl.kernel,
        out_type=jax.ShapeDtypeStruct((NC * n_pad,), jnp.float32),
        mesh=_mesh(),
        compiler_params=_SC_PARAMS,
        scratch_types=[
            pltpu.VMEM((ch,), jnp.int32),
            pltpu.VMEM((ch,), jnp.float32),
            pltpu.VMEM((rps,), jnp.float32),
            pltpu.VMEM_SHARED((n_pad,), jnp.float32),
            pltpu.SemaphoreType.DMA,
        ],
    )
    def k(dst_hbm, out_hbm, dstv, ones, zb, acc, sem):
        c = lax.axis_index("c")
        s = lax.axis_index("s")

        @pl.loop(0, rps // 16)
        def _(i):
            zb[pl.ds(i * 16, 16)] = jnp.zeros((16,), jnp.float32)

        @pl.loop(0, ch // 16)
        def _(i):
            ones[pl.ds(i * 16, 16)] = jnp.ones((16,), jnp.float32)

        pltpu.sync_copy(zb, acc.at[pl.ds(s * rps, rps)])
        plsc.subcore_barrier()

        base = (c * NS + s) * ew

        @pl.loop(0, outer)
        def _(t):
            pltpu.sync_copy(dst_hbm.at[pl.ds(base + t * ch, ch)], dstv)
            pltpu.async_copy(ones, acc.at[dstv], sem, add=True).wait()

        plsc.subcore_barrier()
        pltpu.sync_copy(acc.at[pl.ds(s * rps, rps)], zb)
        pltpu.sync_copy(zb, out_hbm.at[pl.ds(c * n_pad + s * rps, rps)])

    return k(dst_flat).reshape(NC, n_pad)


def _sc_edge_pass(ht, ei4):
    n_pad = ht.shape[0]
    nw, outer, _, ch = ei4.shape
    rps = n_pad // NS
    zrows = rps // 17
    R = 4

    @functools.partial(
        pl.kernel,
        out_type=jax.ShapeDtypeStruct((NC, n_pad, D), jnp.float32),
        mesh=_mesh(),
        compiler_params=_SC_PARAMS,
        scratch_types=[
            pltpu.VMEM((R, 2, ch), jnp.int32),
            pltpu.VMEM((R, ch, D), jnp.float32),
            pltpu.VMEM((zrows, D), jnp.float32),
            pltpu.VMEM_SHARED((n_pad, D), jnp.float32),
            pltpu.SemaphoreType.DMA,
            pltpu.SemaphoreType.DMA,
            pltpu.SemaphoreType.DMA,
        ],
    )
    def k(h_hbm, ei_hbm, out_hbm, idxv, rowsv, zb, acc, semi, semg, sems):
        c = lax.axis_index("c")
        s = lax.axis_index("s")
        w = c * NS + s

        @pl.loop(0, zrows)
        def _(i):
            zb[i, :] = jnp.zeros((D,), jnp.float32)

        @pl.loop(0, rps // zrows)
        def _(i):
            pltpu.sync_copy(zb, acc.at[pl.ds(s * rps + i * zrows, zrows), :])

        plsc.subcore_barrier()

        def fire_idx(t, slot):
            pltpu.async_copy(ei_hbm.at[w, t], idxv.at[slot], semi)

        def wait_idx(slot):
            pltpu.make_async_copy(ei_hbm.at[w, 0], idxv.at[slot], semi).wait()

        def fire_gather(slot):
            pltpu.async_copy(h_hbm.at[idxv.at[slot, 0]], rowsv.at[slot], semg)

        def wait_gather(slot):
            pltpu.make_async_copy(h_hbm.at[idxv.at[slot, 0]], rowsv.at[slot],
                                  semg).wait()

        def fire_scat(slot):
            pltpu.async_copy(rowsv.at[slot], acc.at[idxv.at[slot, 1]], sems,
                             add=True)

        def wait_scat(slot):
            pltpu.make_async_copy(rowsv.at[slot], acc.at[idxv.at[slot, 1]],
                                  sems).wait()

        fire_idx(0, 0)
        fire_idx(1, 1)
        wait_idx(0)
        fire_gather(0)

        @pl.loop(0, outer // R)
        def _(g):
            for r in range(R):
                t = g * R + r

                @pl.when(t >= 2)
                def _():
                    wait_scat((r + 2) % R)

                @pl.when(t < outer - 2)
                def _():
                    fire_idx(t + 2, (r + 2) % R)

                wait_gather(r)
                fire_scat(r)

                @pl.when(t < outer - 1)
                def _():
                    wait_idx((r + 1) % R)
                    fire_gather((r + 1) % R)

        wait_scat((outer - 2) % R)
        wait_scat((outer - 1) % R)

        plsc.subcore_barrier()

        @pl.loop(0, rps // zrows)
        def _(i):
            pltpu.sync_copy(acc.at[pl.ds(s * rps + i * zrows, zrows), :], zb)
            pltpu.sync_copy(zb,
                            out_hbm.at[c, pl.ds(s * rps + i * zrows, zrows), :])

    return k(ht, ei4)


def _tc_matmul(a, w, bias=None):
    n_pad, kdim = a.shape
    f = w.shape[1]
    rb = n_pad // 16

    def body(*refs):
        if bias is None:
            ar, wr, o = refs
            o[...] = jnp.dot(ar[...], wr[...],
                             preferred_element_type=jnp.float32)
        else:
            ar, wr, br, o = refs
            o[...] = jnp.dot(ar[...], wr[...],
                             preferred_element_type=jnp.float32) + br[...]

    in_specs = [
        pl.BlockSpec((rb, kdim), lambda i: (i, 0)),
        pl.BlockSpec((kdim, f), lambda i: (0, 0)),
    ]
    args = [a, w]
    if bias is not None:
        in_specs.append(pl.BlockSpec((1, f), lambda i: (0, 0)))
        args.append(bias)
    return pl.pallas_call(
        body,
        grid=(16,),
        in_specs=in_specs,
        out_specs=pl.BlockSpec((rb, f), lambda i: (i, 0)),
        out_shape=jax.ShapeDtypeStruct((n_pad, f), jnp.float32),
    )(*args)


def _tc_scale(deg_t, h1raw):
    n_pad = h1raw.shape[0]
    rb = n_pad // 16

    def body(dref, href, dout, hout):
        dsum = dref[:, 0:1] + dref[:, 1:2] + 1.0
        dcol = lax.rsqrt(dsum)
        de = jnp.broadcast_to(dcol, (rb, D))
        dout[...] = de
        hout[...] = href[...] * de

    return pl.pallas_call(
        body,
        grid=(16,),
        in_specs=[
            pl.BlockSpec((rb, 2), lambda i: (i, 0)),
            pl.BlockSpec((rb, D), lambda i: (i, 0)),
        ],
        out_specs=[
            pl.BlockSpec((rb, D), lambda i: (i, 0)),
            pl.BlockSpec((rb, D), lambda i: (i, 0)),
        ],
        out_shape=[
            jax.ShapeDtypeStruct((n_pad, D), jnp.float32),
            jax.ShapeDtypeStruct((n_pad, D), jnp.float32),
        ],
    )(deg_t, h1raw)


def _tc_combine(p0p, p1p, htp, dp, be):
    m = p0p.shape[0]
    rb = m // 4

    def body(a, b, h, dd, br, tout, tdout):
        z = (a[...] + b[...] + h[...]) * dd[...] + br[...]
        t = jnp.tanh(z)
        tout[...] = t
        tdout[...] = t * dd[...]

    blk = pl.BlockSpec((rb, 128), lambda i: (i, 0))
    return pl.pallas_call(
        body,
        grid=(4,),
        in_specs=[blk, blk, blk, blk, pl.BlockSpec((1, 128), lambda i: (0, 0))],
        out_specs=[blk, blk],
        out_shape=[
            jax.ShapeDtypeStruct((m, 128), jnp.float32),
            jax.ShapeDtypeStruct((m, 128), jnp.float32),
        ],
    )(p0p, p1p, htp, dp, be)


def _pad_w(w):
    k, f = w.shape
    return jnp.zeros((k, D), jnp.float32).at[:, :f].set(w)


def _pad_b(b):
    bp = jnp.zeros((D,), jnp.float32).at[: b.shape[0]].set(b)
    return jnp.tile(bp, 128 // D).reshape(1, 128)


def kernel(x, edge_index, W1, b1, W2, b2, W3, b3, Wc, bc):
    n = x.shape[0]
    e = edge_index.shape[1]
    n_pad = -(-(n + 1) // 128) * 128
    m = n_pad * D // 128

    src = edge_index[0].astype(jnp.int32)
    dst = edge_index[1].astype(jnp.int32)

    rows = -(-e // LANES)
    rows_w = -(-rows // (NC * NS))
    rows_w = -(-rows_w // SB) * SB
    rows_p = rows_w * NC * NS
    pad = rows_p * LANES - e
    scratch_rows = n_pad - n
    src_p = jnp.concatenate([src, jnp.zeros((pad,), jnp.int32)])
    dst_p = jnp.concatenate(
        [dst, n + (jnp.arange(pad, dtype=jnp.int32) % scratch_rows)])

    nw = NC * NS
    ew = rows_w * LANES
    ch = 320
    outer = ew // ch
    ei4 = jnp.stack([src_p.reshape(nw, outer, ch),
                     dst_p.reshape(nw, outer, ch)], axis=2)

    xp = jnp.pad(x, ((0, n_pad - n), (0, 0)))
    w1p, w2p, w3p = _pad_w(W1), _pad_w(W2), _pad_w(W3)
    wcp = _pad_w(Wc)
    be1, be2, be3 = _pad_b(b1), _pad_b(b2), _pad_b(b3)

    h1raw = _tc_matmul(xp, w1p)
    degp = _sc_degree(dst_p, n_pad)

    deg_t = jnp.transpose(degp)
    dinv_e, h1t = _tc_scale(deg_t, h1raw)
    dp = dinv_e.reshape(m, 128)

    ht = h1t
    t_packed = None
    for wnext, be in ((w2p, be1), (w3p, be2), (None, be3)):
        p = _sc_edge_pass(ht, ei4)
        t_packed, td_packed = _tc_combine(
            p[0].reshape(m, 128), p[1].reshape(m, 128),
            ht.reshape(m, 128), dp, be)
        if wnext is not None:
            ht = _tc_matmul(td_packed.reshape(n_pad, D), wnext)

    t3 = t_packed.reshape(n_pad, D)
    bcp = jnp.zeros((1, D), jnp.float32).at[0, : bc.shape[0]].set(bc)
    outp = _tc_matmul(t3, wcp, bcp)

    return (outp[:n, : Wc.shape[1]], t3[:n, : W3.shape[1]])


# speedup vs baseline: 1.3161x; 1.3161x over previous
import functools

import jax
import jax.numpy as jnp
from jax import lax
from jax.experimental import pallas as pl
from jax.experimental.pallas import tpu as pltpu
from jax.experimental.pallas import tpu_sc as plsc

NC = 2
NS = 16
LANES = 128
SB = 16
SBE = 8
D = 16


def _mesh():
    return plsc.VectorSubcoreMesh(core_axis_name="c", subcore_axis_name="s")


_SC_PARAMS = pltpu.CompilerParams(use_tc_tiling_on_sc=False)


def _sc_degree(dst_flat, n_pad):
    ep = dst_flat.shape[0]
    ew = ep // (NC * NS)
    ch = SB * LANES
    outer = ew // ch
    rps = n_pad // NS

    @functools.partial(
        pl.kernel,
        out_type=jax.ShapeDtypeStruct((NC * n_pad,), jnp.float32),
        mesh=_mesh(),
        compiler_params=_SC_PARAMS,
        scratch_types=[
            pltpu.VMEM((ch,), jnp.int32),
            pltpu.VMEM((ch,), jnp.float32),
            pltpu.VMEM((rps,), jnp.float32),
            pltpu.VMEM_SHARED((n_pad,), jnp.float32),
            pltpu.SemaphoreType.DMA,
        ],
    )
    def k(dst_hbm, out_hbm, dstv, ones, zb, acc, sem):
        c = lax.axis_index("c")
        s = lax.axis_index("s")

        @pl.loop(0, rps // 16)
        def _(i):
            zb[pl.ds(i * 16, 16)] = jnp.zeros((16,), jnp.float32)

        @pl.loop(0, ch // 16)
        def _(i):
            ones[pl.ds(i * 16, 16)] = jnp.ones((16,), jnp.float32)

        pltpu.sync_copy(zb, acc.at[pl.ds(s * rps, rps)])
        plsc.subcore_barrier()

        base = (c * NS + s) * ew

        @pl.loop(0, outer)
        def _(t):
            pltpu.sync_copy(dst_hbm.at[pl.ds(base + t * ch, ch)], dstv)
            pltpu.async_copy(ones, acc.at[dstv], sem, add=True).wait()

        plsc.subcore_barrier()
        pltpu.sync_copy(acc.at[pl.ds(s * rps, rps)], zb)
        pltpu.sync_copy(zb, out_hbm.at[pl.ds(c * n_pad + s * rps, rps)])

    return k(dst_flat).reshape(NC, n_pad)


def _sc_edge_pass(ht, ei4):
    n_pad = ht.shape[0]
    nw, outer, _, ch = ei4.shape
    rps = n_pad // NS
    zrows = rps // 17
    R = 4

    @functools.partial(
        pl.kernel,
        out_type=jax.ShapeDtypeStruct((NC, n_pad, D), jnp.float32),
        mesh=_mesh(),
        compiler_params=_SC_PARAMS,
        scratch_types=[
            pltpu.VMEM((R, 2, ch), jnp.int32),
            pltpu.VMEM((R, ch, D), jnp.float32),
            pltpu.VMEM((zrows, D), jnp.float32),
            pltpu.VMEM_SHARED((n_pad, D), jnp.float32),
            pltpu.SemaphoreType.DMA,
            pltpu.SemaphoreType.DMA,
            pltpu.SemaphoreType.DMA,
        ],
    )
    def k(h_hbm, ei_hbm, out_hbm, idxv, rowsv, zb, acc, semi, semg, sems):
        c = lax.axis_index("c")
        s = lax.axis_index("s")
        w = c * NS + s

        @pl.loop(0, zrows)
        def _(i):
            zb[i, :] = jnp.zeros((D,), jnp.float32)

        @pl.loop(0, rps // zrows)
        def _(i):
            pltpu.sync_copy(zb, acc.at[pl.ds(s * rps + i * zrows, zrows), :])

        plsc.subcore_barrier()

        def fire_idx(t, slot):
            pltpu.async_copy(ei_hbm.at[w, t], idxv.at[slot], semi)

        def wait_idx(slot):
            pltpu.make_async_copy(ei_hbm.at[w, 0], idxv.at[slot], semi).wait()

        def fire_gather(slot):
            pltpu.async_copy(h_hbm.at[idxv.at[slot, 0]], rowsv.at[slot], semg)

        def wait_gather(slot):
            pltpu.make_async_copy(h_hbm.at[idxv.at[slot, 0]], rowsv.at[slot],
                                  semg).wait()

        def fire_scat(slot):
            pltpu.async_copy(rowsv.at[slot], acc.at[idxv.at[slot, 1]], sems,
                             add=True)

        def wait_scat(slot):
            pltpu.make_async_copy(rowsv.at[slot], acc.at[idxv.at[slot, 1]],
                                  sems).wait()

        fire_idx(0, 0)
        fire_idx(1, 1)
        wait_idx(0)
        fire_gather(0)

        @pl.loop(0, outer // R)
        def _(g):
            for r in range(R):
                t = g * R + r

                @pl.when(t >= 2)
                def _():
                    wait_scat((r + 2) % R)

                @pl.when(t < outer - 2)
                def _():
                    fire_idx(t + 2, (r + 2) % R)

                wait_gather(r)
                fire_scat(r)

                @pl.when(t < outer - 1)
                def _():
                    wait_idx((r + 1) % R)
                    fire_gather((r + 1) % R)

        wait_scat((outer - 2) % R)
        wait_scat((outer - 1) % R)

        plsc.subcore_barrier()

        @pl.loop(0, rps // zrows)
        def _(i):
            pltpu.sync_copy(acc.at[pl.ds(s * rps + i * zrows, zrows), :], zb)
            pltpu.sync_copy(zb,
                            out_hbm.at[c, pl.ds(s * rps + i * zrows, zrows), :])

    return k(ht, ei4)


def _sc_edge_pass_sp(ht, ei4):
    n_pad = ht.shape[0]
    d8 = ht.shape[1]
    nw, outer, _, ch = ei4.shape
    rps = n_pad // NS
    zrows = rps // 17
    R = 4

    @functools.partial(
        pl.kernel,
        out_type=jax.ShapeDtypeStruct((NC, n_pad, d8), jnp.float32),
        mesh=_mesh(),
        compiler_params=_SC_PARAMS,
        scratch_types=[
            pltpu.VMEM((R, 2, ch), jnp.int32),
            pltpu.VMEM((R, ch, d8), jnp.float32),
            pltpu.VMEM((zrows, d8), jnp.float32),
            pltpu.VMEM_SHARED((n_pad, d8), jnp.float32),
            pltpu.VMEM_SHARED((n_pad, d8), jnp.float32),
            pltpu.SemaphoreType.DMA,
            pltpu.SemaphoreType.DMA,
            pltpu.SemaphoreType.DMA,
        ],
    )
    def k(h_hbm, ei_hbm, out_hbm, idxv, rowsv, zb, tbl, acc, semi, semg, sems):
        c = lax.axis_index("c")
        s = lax.axis_index("s")
        w = c * NS + s

        pltpu.sync_copy(h_hbm.at[pl.ds(s * rps, rps), :],
                        tbl.at[pl.ds(s * rps, rps), :])

        @pl.loop(0, zrows)
        def _(i):
            zb[i, :] = jnp.zeros((d8,), jnp.float32)

        @pl.loop(0, rps // zrows)
        def _(i):
            pltpu.sync_copy(zb, acc.at[pl.ds(s * rps + i * zrows, zrows), :])

        plsc.subcore_barrier()

        def fire_idx(t, slot):
            pltpu.async_copy(ei_hbm.at[w, t], idxv.at[slot], semi)

        def wait_idx(slot):
            pltpu.make_async_copy(ei_hbm.at[w, 0], idxv.at[slot], semi).wait()

        def fire_gather(slot):
            pltpu.async_copy(tbl.at[idxv.at[slot, 0]], rowsv.at[slot], semg)

        def wait_gather(slot):
            pltpu.make_async_copy(tbl.at[idxv.at[slot, 0]], rowsv.at[slot],
                                  semg).wait()

        def fire_scat(slot):
            pltpu.async_copy(rowsv.at[slot], acc.at[idxv.at[slot, 1]], sems,
                             add=True)

        def wait_scat(slot):
            pltpu.make_async_copy(rowsv.at[slot], acc.at[idxv.at[slot, 1]],
                                  sems).wait()

        fire_idx(0, 0)
        fire_idx(1, 1)
        wait_idx(0)
        fire_gather(0)

        @pl.loop(0, outer // R)
        def _(g):
            for r in range(R):
                t = g * R + r

                @pl.when(t >= 2)
                def _():
                    wait_scat((r + 2) % R)

                @pl.when(t < outer - 2)
                def _():
                    fire_idx(t + 2, (r + 2) % R)

                wait_gather(r)
                fire_scat(r)

                @pl.when(t < outer - 1)
                def _():
                    wait_idx((r + 1) % R)
                    fire_gather((r + 1) % R)

        wait_scat((outer - 2) % R)
        wait_scat((outer - 1) % R)

        plsc.subcore_barrier()

        @pl.loop(0, rps // zrows)
        def _(i):
            pltpu.sync_copy(acc.at[pl.ds(s * rps + i * zrows, zrows), :], zb)
            pltpu.sync_copy(zb,
                            out_hbm.at[c, pl.ds(s * rps + i * zrows, zrows), :])

    return k(ht, ei4)


def _tc_matmul(a, w, bias=None):
    n_pad, kdim = a.shape
    f = w.shape[1]
    rb = n_pad // 16

    def body(*refs):
        if bias is None:
            ar, wr, o = refs
            o[...] = jnp.dot(ar[...], wr[...],
                             preferred_element_type=jnp.float32)
        else:
            ar, wr, br, o = refs
            o[...] = jnp.dot(ar[...], wr[...],
                             preferred_element_type=jnp.float32) + br[...]

    in_specs = [
        pl.BlockSpec((rb, kdim), lambda i: (i, 0)),
        pl.BlockSpec((kdim, f), lambda i: (0, 0)),
    ]
    args = [a, w]
    if bias is not None:
        in_specs.append(pl.BlockSpec((1, f), lambda i: (0, 0)))
        args.append(bias)
    return pl.pallas_call(
        body,
        grid=(16,),
        in_specs=in_specs,
        out_specs=pl.BlockSpec((rb, f), lambda i: (i, 0)),
        out_shape=jax.ShapeDtypeStruct((n_pad, f), jnp.float32),
    )(*args)


def _tc_scale(deg_t, h1raw):
    n_pad = h1raw.shape[0]
    rb = n_pad // 16

    def body(dref, href, dout, hout):
        dsum = dref[:, 0:1] + dref[:, 1:2] + 1.0
        dcol = lax.rsqrt(dsum)
        de = jnp.broadcast_to(dcol, (rb, D))
        dout[...] = de
        hout[...] = href[...] * de

    return pl.pallas_call(
        body,
        grid=(16,),
        in_specs=[
            pl.BlockSpec((rb, 2), lambda i: (i, 0)),
            pl.BlockSpec((rb, D), lambda i: (i, 0)),
        ],
        out_specs=[
            pl.BlockSpec((rb, D), lambda i: (i, 0)),
            pl.BlockSpec((rb, D), lambda i: (i, 0)),
        ],
        out_shape=[
            jax.ShapeDtypeStruct((n_pad, D), jnp.float32),
            jax.ShapeDtypeStruct((n_pad, D), jnp.float32),
        ],
    )(deg_t, h1raw)


def _tc_combine(p0p, p1p, htp, dp, be):
    m = p0p.shape[0]
    grid = next(g for g in (4, 2, 1) if (m // g) % 8 == 0)
    rb = m // grid

    def body(a, b, h, dd, br, tout, tdout):
        z = (a[...] + b[...] + h[...]) * dd[...] + br[...]
        t = jnp.tanh(z)
        tout[...] = t
        tdout[...] = t * dd[...]

    blk = pl.BlockSpec((rb, 128), lambda i: (i, 0))
    return pl.pallas_call(
        body,
        grid=(grid,),
        in_specs=[blk, blk, blk, blk, pl.BlockSpec((1, 128), lambda i: (0, 0))],
        out_specs=[blk, blk],
        out_shape=[
            jax.ShapeDtypeStruct((m, 128), jnp.float32),
            jax.ShapeDtypeStruct((m, 128), jnp.float32),
        ],
    )(p0p, p1p, htp, dp, be)


def _pad_w(w, width=D):
    k, f = w.shape
    return jnp.zeros((k, width), jnp.float32).at[:, :f].set(w)


def _pad_b(b, width=D):
    bp = jnp.zeros((width,), jnp.float32).at[: b.shape[0]].set(b)
    return jnp.tile(bp, 128 // width).reshape(1, 128)


def kernel(x, edge_index, W1, b1, W2, b2, W3, b3, Wc, bc):
    n = x.shape[0]
    e = edge_index.shape[1]
    n_pad = -(-(n + 1) // 128) * 128
    m = n_pad * D // 128

    src = edge_index[0].astype(jnp.int32)
    dst = edge_index[1].astype(jnp.int32)

    rows = -(-e // LANES)
    rows_w = -(-rows // (NC * NS))
    rows_w = -(-rows_w // SB) * SB
    rows_p = rows_w * NC * NS
    pad = rows_p * LANES - e
    scratch_rows = n_pad - n
    src_p = jnp.concatenate([src, jnp.zeros((pad,), jnp.int32)])
    dst_p = jnp.concatenate(
        [dst, n + (jnp.arange(pad, dtype=jnp.int32) % scratch_rows)])

    nw = NC * NS
    ew = rows_w * LANES
    ch = 320
    outer = ew // ch
    ei4 = jnp.stack([src_p.reshape(nw, outer, ch),
                     dst_p.reshape(nw, outer, ch)], axis=2)

    d8 = 8
    m8 = n_pad * d8 // 128
    xp = jnp.pad(x, ((0, n_pad - n), (0, 0)))
    w1p = _pad_w(W1)
    w2p, w3p, wcp = _pad_w(W2, d8), _pad_w(W3, d8), _pad_w(Wc, d8)
    be1 = _pad_b(b1)
    be2, be3 = _pad_b(b2, d8), _pad_b(b3, d8)

    h1raw = _tc_matmul(xp, w1p)
    degp = _sc_degree(dst_p, n_pad)

    deg_t = jnp.transpose(degp)
    dinv_e, h1t = _tc_scale(deg_t, h1raw)
    dp = dinv_e.reshape(m, 128)
    dp8 = dinv_e[:, :d8].reshape(m8, 128)

    p = _sc_edge_pass(h1t, ei4)
    _, td1 = _tc_combine(p[0].reshape(m, 128), p[1].reshape(m, 128),
                         h1t.reshape(m, 128), dp, be1)
    ht2 = _tc_matmul(td1.reshape(n_pad, D), w2p)

    p = _sc_edge_pass_sp(ht2, ei4)
    _, td2 = _tc_combine(p[0].reshape(m8, 128), p[1].reshape(m8, 128),
                         ht2.reshape(m8, 128), dp8, be2)
    ht3 = _tc_matmul(td2.reshape(n_pad, d8), w3p)

    p = _sc_edge_pass_sp(ht3, ei4)
    t3p, _ = _tc_combine(p[0].reshape(m8, 128), p[1].reshape(m8, 128),
                         ht3.reshape(m8, 128), dp8, be3)

    t3 = t3p.reshape(n_pad, d8)
    bcp = jnp.zeros((1, d8), jnp.float32).at[0, : bc.shape[0]].set(bc)
    outp = _tc_matmul(t3, wcp, bcp)

    return (outp[:n, : Wc.shape[1]], t3[:n, : W3.shape[1]])


# speedup vs baseline: 1.6165x; 1.2282x over previous
import functools

import jax
import jax.numpy as jnp
from jax import lax
from jax.experimental import pallas as pl
from jax.experimental.pallas import tpu as pltpu
from jax.experimental.pallas import tpu_sc as plsc

NC = 2
NS = 16
LANES = 128
SB = 16
SBE = 8
D = 16


def _mesh():
    return plsc.VectorSubcoreMesh(core_axis_name="c", subcore_axis_name="s")


_SC_PARAMS = pltpu.CompilerParams(use_tc_tiling_on_sc=False)


def _sc_degree(dst_flat, n_pad):
    ep = dst_flat.shape[0]
    ew = ep // (NC * NS)
    ch = SB * LANES
    outer = ew // ch
    rps = n_pad // NS

    @functools.partial(
        pl.kernel,
        out_type=jax.ShapeDtypeStruct((NC * n_pad,), jnp.float32),
        mesh=_mesh(),
        compiler_params=_SC_PARAMS,
        scratch_types=[
            pltpu.VMEM((ch,), jnp.int32),
            pltpu.VMEM((ch,), jnp.float32),
            pltpu.VMEM((rps,), jnp.float32),
            pltpu.VMEM_SHARED((n_pad,), jnp.float32),
            pltpu.SemaphoreType.DMA,
        ],
    )
    def k(dst_hbm, out_hbm, dstv, ones, zb, acc, sem):
        c = lax.axis_index("c")
        s = lax.axis_index("s")

        @pl.loop(0, rps // 16)
        def _(i):
            zb[pl.ds(i * 16, 16)] = jnp.zeros((16,), jnp.float32)

        @pl.loop(0, ch // 16)
        def _(i):
            ones[pl.ds(i * 16, 16)] = jnp.ones((16,), jnp.float32)

        pltpu.sync_copy(zb, acc.at[pl.ds(s * rps, rps)])
        plsc.subcore_barrier()

        base = (c * NS + s) * ew

        @pl.loop(0, outer)
        def _(t):
            pltpu.sync_copy(dst_hbm.at[pl.ds(base + t * ch, ch)], dstv)
            pltpu.async_copy(ones, acc.at[dstv], sem, add=True).wait()

        plsc.subcore_barrier()
        pltpu.sync_copy(acc.at[pl.ds(s * rps, rps)], zb)
        pltpu.sync_copy(zb, out_hbm.at[pl.ds(c * n_pad + s * rps, rps)])

    return k(dst_flat).reshape(NC, n_pad)


def _sc_edge_pass(ht, ei4):
    n_pad = ht.shape[0]
    nw, outer, _, ch = ei4.shape
    rps = n_pad // NS
    zrows = rps // 17
    R = 4

    @functools.partial(
        pl.kernel,
        out_type=jax.ShapeDtypeStruct((NC, n_pad, D), jnp.float32),
        mesh=_mesh(),
        compiler_params=_SC_PARAMS,
        scratch_types=[
            pltpu.VMEM((R, 2, ch), jnp.int32),
            pltpu.VMEM((R, ch, D), jnp.float32),
            pltpu.VMEM((zrows, D), jnp.float32),
            pltpu.VMEM_SHARED((n_pad, D), jnp.float32),
            pltpu.SemaphoreType.DMA,
            pltpu.SemaphoreType.DMA,
            pltpu.SemaphoreType.DMA,
        ],
    )
    def k(h_hbm, ei_hbm, out_hbm, idxv, rowsv, zb, acc, semi, semg, sems):
        c = lax.axis_index("c")
        s = lax.axis_index("s")
        w = c * NS + s

        @pl.loop(0, zrows)
        def _(i):
            zb[i, :] = jnp.zeros((D,), jnp.float32)

        @pl.loop(0, rps // zrows)
        def _(i):
            pltpu.sync_copy(zb, acc.at[pl.ds(s * rps + i * zrows, zrows), :])

        plsc.subcore_barrier()

        def fire_idx(t, slot):
            pltpu.async_copy(ei_hbm.at[w, t], idxv.at[slot], semi)

        def wait_idx(slot):
            pltpu.make_async_copy(ei_hbm.at[w, 0], idxv.at[slot], semi).wait()

        def fire_gather(slot):
            pltpu.async_copy(h_hbm.at[idxv.at[slot, 0]], rowsv.at[slot], semg)

        def wait_gather(slot):
            pltpu.make_async_copy(h_hbm.at[idxv.at[slot, 0]], rowsv.at[slot],
                                  semg).wait()

        def fire_scat(slot):
            pltpu.async_copy(rowsv.at[slot], acc.at[idxv.at[slot, 1]], sems,
                             add=True)

        def wait_scat(slot):
            pltpu.make_async_copy(rowsv.at[slot], acc.at[idxv.at[slot, 1]],
                                  sems).wait()

        fire_idx(0, 0)
        fire_idx(1, 1)
        wait_idx(0)
        fire_gather(0)

        @pl.loop(0, outer // R)
        def _(g):
            for r in range(R):
                t = g * R + r

                @pl.when(t >= 2)
                def _():
                    wait_scat((r + 2) % R)

                @pl.when(t < outer - 2)
                def _():
                    fire_idx(t + 2, (r + 2) % R)

                wait_gather(r)
                fire_scat(r)

                @pl.when(t < outer - 1)
                def _():
                    wait_idx((r + 1) % R)
                    fire_gather((r + 1) % R)

        wait_scat((outer - 2) % R)
        wait_scat((outer - 1) % R)

        plsc.subcore_barrier()

        @pl.loop(0, rps // zrows)
        def _(i):
            pltpu.sync_copy(acc.at[pl.ds(s * rps + i * zrows, zrows), :], zb)
            pltpu.sync_copy(zb,
                            out_hbm.at[c, pl.ds(s * rps + i * zrows, zrows), :])

    return k(ht, ei4)


def _sc_edge_pass_sp(ht, ei4):
    n_pad = ht.shape[0]
    d8 = ht.shape[1]
    nw, outer, _, ch = ei4.shape
    rps = n_pad // NS
    zrows = rps // 17
    R = 4

    @functools.partial(
        pl.kernel,
        out_type=jax.ShapeDtypeStruct((NC, n_pad, d8), jnp.float32),
        mesh=_mesh(),
        compiler_params=_SC_PARAMS,
        scratch_types=[
            pltpu.VMEM((R, 2, ch), jnp.int32),
            pltpu.VMEM((R, ch, d8), jnp.float32),
            pltpu.VMEM((zrows, d8), jnp.float32),
            pltpu.VMEM_SHARED((n_pad, d8), jnp.float32),
            pltpu.VMEM_SHARED((n_pad, d8), jnp.float32),
            pltpu.SemaphoreType.DMA,
            pltpu.SemaphoreType.DMA,
            pltpu.SemaphoreType.DMA,
        ],
    )
    def k(h_hbm, ei_hbm, out_hbm, idxv, rowsv, zb, tbl, acc, semi, semg, sems):
        c = lax.axis_index("c")
        s = lax.axis_index("s")
        w = c * NS + s

        pltpu.sync_copy(h_hbm.at[pl.ds(s * rps, rps), :],
                        tbl.at[pl.ds(s * rps, rps), :])

        @pl.loop(0, zrows)
        def _(i):
            zb[i, :] = jnp.zeros((d8,), jnp.float32)

        @pl.loop(0, rps // zrows)
        def _(i):
            pltpu.sync_copy(zb, acc.at[pl.ds(s * rps + i * zrows, zrows), :])

        plsc.subcore_barrier()

        def fire_idx(t, slot):
            pltpu.async_copy(ei_hbm.at[w, t], idxv.at[slot], semi)

        def wait_idx(slot):
            pltpu.make_async_copy(ei_hbm.at[w, 0], idxv.at[slot], semi).wait()

        def fire_gather(slot):
            pltpu.async_copy(tbl.at[idxv.at[slot, 0]], rowsv.at[slot], semg)

        def wait_gather(slot):
            pltpu.make_async_copy(tbl.at[idxv.at[slot, 0]], rowsv.at[slot],
                                  semg).wait()

        def fire_scat(slot):
            pltpu.async_copy(rowsv.at[slot], acc.at[idxv.at[slot, 1]], sems,
                             add=True)

        def wait_scat(slot):
            pltpu.make_async_copy(rowsv.at[slot], acc.at[idxv.at[slot, 1]],
                                  sems).wait()

        fire_idx(0, 0)
        fire_idx(1, 1)
        wait_idx(0)
        fire_gather(0)

        @pl.loop(0, outer // R)
        def _(g):
            for r in range(R):
                t = g * R + r

                @pl.when(t >= 2)
                def _():
                    wait_scat((r + 2) % R)

                @pl.when(t < outer - 2)
                def _():
                    fire_idx(t + 2, (r + 2) % R)

                wait_gather(r)
                fire_scat(r)

                @pl.when(t < outer - 1)
                def _():
                    wait_idx((r + 1) % R)
                    fire_gather((r + 1) % R)

        wait_scat((outer - 2) % R)
        wait_scat((outer - 1) % R)

        plsc.subcore_barrier()

        @pl.loop(0, rps // zrows)
        def _(i):
            pltpu.sync_copy(acc.at[pl.ds(s * rps + i * zrows, zrows), :], zb)
            pltpu.sync_copy(zb,
                            out_hbm.at[c, pl.ds(s * rps + i * zrows, zrows), :])

    return k(ht, ei4)


def _tc_matmul(a, w, bias=None):
    n_pad, kdim = a.shape
    f = w.shape[1]
    rb = n_pad // 16

    def body(*refs):
        if bias is None:
            ar, wr, o = refs
            o[...] = jnp.dot(ar[...], wr[...],
                             preferred_element_type=jnp.float32)
        else:
            ar, wr, br, o = refs
            o[...] = jnp.dot(ar[...], wr[...],
                             preferred_element_type=jnp.float32) + br[...]

    in_specs = [
        pl.BlockSpec((rb, kdim), lambda i: (i, 0)),
        pl.BlockSpec((kdim, f), lambda i: (0, 0)),
    ]
    args = [a, w]
    if bias is not None:
        in_specs.append(pl.BlockSpec((1, f), lambda i: (0, 0)))
        args.append(bias)
    return pl.pallas_call(
        body,
        grid=(16,),
        in_specs=in_specs,
        out_specs=pl.BlockSpec((rb, f), lambda i: (i, 0)),
        out_shape=jax.ShapeDtypeStruct((n_pad, f), jnp.float32),
    )(*args)


def _tc_scale(deg_t, h1raw):
    n_pad = h1raw.shape[0]
    rb = n_pad // 16

    def body(dref, href, dout, hout):
        dsum = dref[:, 0:1] + dref[:, 1:2] + 1.0
        dcol = lax.rsqrt(dsum)
        de = jnp.broadcast_to(dcol, (rb, D))
        dout[...] = de
        hout[...] = href[...] * de

    return pl.pallas_call(
        body,
        grid=(16,),
        in_specs=[
            pl.BlockSpec((rb, 2), lambda i: (i, 0)),
            pl.BlockSpec((rb, D), lambda i: (i, 0)),
        ],
        out_specs=[
            pl.BlockSpec((rb, D), lambda i: (i, 0)),
            pl.BlockSpec((rb, D), lambda i: (i, 0)),
        ],
        out_shape=[
            jax.ShapeDtypeStruct((n_pad, D), jnp.float32),
            jax.ShapeDtypeStruct((n_pad, D), jnp.float32),
        ],
    )(deg_t, h1raw)


def _tc_combine(p, htf, dinv_e, be):
    n_pad, dd = htf.shape
    rb = n_pad // 16

    def body(pref, href, dref, bref, tout, tdout):
        de = dref[:, :dd]
        z = (pref[0] + pref[1] + href[...]) * de + bref[...]
        t = jnp.tanh(z)
        tout[...] = t
        tdout[...] = t * de

    return pl.pallas_call(
        body,
        grid=(16,),
        in_specs=[
            pl.BlockSpec((2, rb, dd), lambda i: (0, i, 0)),
            pl.BlockSpec((rb, dd), lambda i: (i, 0)),
            pl.BlockSpec((rb, D), lambda i: (i, 0)),
            pl.BlockSpec((1, dd), lambda i: (0, 0)),
        ],
        out_specs=[
            pl.BlockSpec((rb, dd), lambda i: (i, 0)),
            pl.BlockSpec((rb, dd), lambda i: (i, 0)),
        ],
        out_shape=[
            jax.ShapeDtypeStruct((n_pad, dd), jnp.float32),
            jax.ShapeDtypeStruct((n_pad, dd), jnp.float32),
        ],
    )(p, htf, dinv_e, be)


def _tc_final(t3f, wc, bc, n, hw):
    n_pad, dd = t3f.shape
    fo = wc.shape[1]
    grid = 10
    rb = n // grid

    def body(tref, wref, bref, oout, hout):
        t = tref[...]
        oout[...] = jnp.dot(t, wref[...],
                            preferred_element_type=jnp.float32) + bref[...]
        hout[...] = t[:, :hw]

    return pl.pallas_call(
        body,
        grid=(grid,),
        in_specs=[
            pl.BlockSpec((rb, dd), lambda i: (i, 0)),
            pl.BlockSpec((dd, fo), lambda i: (0, 0)),
            pl.BlockSpec((1, fo), lambda i: (0, 0)),
        ],
        out_specs=[
            pl.BlockSpec((rb, fo), lambda i: (i, 0)),
            pl.BlockSpec((rb, hw), lambda i: (i, 0)),
        ],
        out_shape=[
            jax.ShapeDtypeStruct((n, fo), jnp.float32),
            jax.ShapeDtypeStruct((n, hw), jnp.float32),
        ],
    )(t3f, wc, bc)


def _pad_w(w, width=D):
    k, f = w.shape
    return jnp.zeros((k, width), jnp.float32).at[:, :f].set(w)


def _pad_b(b, width=D):
    return jnp.zeros((1, width), jnp.float32).at[0, : b.shape[0]].set(b)


def kernel(x, edge_index, W1, b1, W2, b2, W3, b3, Wc, bc):
    n = x.shape[0]
    e = edge_index.shape[1]
    n_pad = -(-(n + 1) // 128) * 128
    m = n_pad * D // 128

    src = edge_index[0].astype(jnp.int32)
    dst = edge_index[1].astype(jnp.int32)

    rows = -(-e // LANES)
    rows_w = -(-rows // (NC * NS))
    rows_w = -(-rows_w // SB) * SB
    rows_p = rows_w * NC * NS
    pad = rows_p * LANES - e
    scratch_rows = n_pad - n
    src_p = jnp.concatenate([src, jnp.zeros((pad,), jnp.int32)])
    dst_p = jnp.concatenate(
        [dst, n + (jnp.arange(pad, dtype=jnp.int32) % scratch_rows)])

    nw = NC * NS
    ew = rows_w * LANES
    ch = 320
    outer = ew // ch
    ei4 = jnp.stack([src_p.reshape(nw, outer, ch),
                     dst_p.reshape(nw, outer, ch)], axis=2)

    d8 = 8
    xp = jnp.pad(x, ((0, n_pad - n), (0, 0)))
    w1p = _pad_w(W1)
    w2p, w3p = _pad_w(W2, d8), _pad_w(W3, d8)
    wcp = jnp.zeros((d8, Wc.shape[1]), jnp.float32).at[: Wc.shape[0]].set(Wc)
    be1 = _pad_b(b1)
    be2, be3 = _pad_b(b2, d8), _pad_b(b3, d8)
    bcp = bc[None, :]

    h1raw = _tc_matmul(xp, w1p)
    degp = _sc_degree(dst_p, n_pad)

    deg_t = jnp.transpose(degp)
    dinv_e, h1t = _tc_scale(deg_t, h1raw)

    p = _sc_edge_pass(h1t, ei4)
    _, td1 = _tc_combine(p, h1t, dinv_e, be1)
    ht2 = _tc_matmul(td1, w2p)

    p = _sc_edge_pass_sp(ht2, ei4)
    _, td2 = _tc_combine(p, ht2, dinv_e, be2)
    ht3 = _tc_matmul(td2, w3p)

    p = _sc_edge_pass_sp(ht3, ei4)
    t3, _ = _tc_combine(p, ht3, dinv_e, be3)

    out, h = _tc_final(t3, wcp, bcp, n, W3.shape[1])
    return (out, h)


# speedup vs baseline: 1.7378x; 1.0750x over previous
import functools

import jax
import jax.numpy as jnp
from jax import lax
from jax.experimental import pallas as pl
from jax.experimental.pallas import tpu as pltpu
from jax.experimental.pallas import tpu_sc as plsc

NC = 2
NS = 16
LANES = 128
SB = 16
SBE = 8
D = 16


def _mesh():
    return plsc.VectorSubcoreMesh(core_axis_name="c", subcore_axis_name="s")


_SC_PARAMS = pltpu.CompilerParams(use_tc_tiling_on_sc=False)


def _sc_degree(dst_flat, n_pad):
    ep = dst_flat.shape[0]
    ew = ep // (NC * NS)
    ch = SB * LANES
    outer = ew // ch
    rps = n_pad // NS

    @functools.partial(
        pl.kernel,
        out_type=jax.ShapeDtypeStruct((NC * n_pad,), jnp.float32),
        mesh=_mesh(),
        compiler_params=_SC_PARAMS,
        scratch_types=[
            pltpu.VMEM((ch,), jnp.int32),
            pltpu.VMEM((ch,), jnp.float32),
            pltpu.VMEM((rps,), jnp.float32),
            pltpu.VMEM_SHARED((n_pad,), jnp.float32),
            pltpu.SemaphoreType.DMA,
        ],
    )
    def k(dst_hbm, out_hbm, dstv, ones, zb, acc, sem):
        c = lax.axis_index("c")
        s = lax.axis_index("s")

        @pl.loop(0, rps // 16)
        def _(i):
            zb[pl.ds(i * 16, 16)] = jnp.zeros((16,), jnp.float32)

        @pl.loop(0, ch // 16)
        def _(i):
            ones[pl.ds(i * 16, 16)] = jnp.ones((16,), jnp.float32)

        pltpu.sync_copy(zb, acc.at[pl.ds(s * rps, rps)])
        plsc.subcore_barrier()

        base = (c * NS + s) * ew

        @pl.loop(0, outer)
        def _(t):
            pltpu.sync_copy(dst_hbm.at[pl.ds(base + t * ch, ch)], dstv)
            pltpu.async_copy(ones, acc.at[dstv], sem, add=True).wait()

        plsc.subcore_barrier()
        pltpu.sync_copy(acc.at[pl.ds(s * rps, rps)], zb)
        pltpu.sync_copy(zb, out_hbm.at[pl.ds(c * n_pad + s * rps, rps)])

    return k(dst_flat).reshape(NC, n_pad)


def _sc_edge_pass(ht, ei4):
    n_pad = ht.shape[0]
    nw, outer, _, ch = ei4.shape
    rps = n_pad // NS
    zrows = rps // 17
    R = 4

    @functools.partial(
        pl.kernel,
        out_type=jax.ShapeDtypeStruct((NC, n_pad, D), jnp.float32),
        mesh=_mesh(),
        compiler_params=_SC_PARAMS,
        scratch_types=[
            pltpu.VMEM((R, 2, ch), jnp.int32),
            pltpu.VMEM((R, ch, D), jnp.float32),
            pltpu.VMEM((zrows, D), jnp.float32),
            pltpu.VMEM_SHARED((n_pad, D), jnp.float32),
            pltpu.SemaphoreType.DMA,
            pltpu.SemaphoreType.DMA,
            pltpu.SemaphoreType.DMA,
        ],
    )
    def k(h_hbm, ei_hbm, out_hbm, idxv, rowsv, zb, acc, semi, semg, sems):
        c = lax.axis_index("c")
        s = lax.axis_index("s")
        w = c * NS + s

        @pl.loop(0, zrows)
        def _(i):
            zb[i, :] = jnp.zeros((D,), jnp.float32)

        @pl.loop(0, rps // zrows)
        def _(i):
            pltpu.sync_copy(zb, acc.at[pl.ds(s * rps + i * zrows, zrows), :])

        plsc.subcore_barrier()

        def fire_idx(t, slot):
            pltpu.async_copy(ei_hbm.at[w, t], idxv.at[slot], semi)

        def wait_idx(slot):
            pltpu.make_async_copy(ei_hbm.at[w, 0], idxv.at[slot], semi).wait()

        def fire_gather(slot):
            pltpu.async_copy(h_hbm.at[idxv.at[slot, 0]], rowsv.at[slot], semg)

        def wait_gather(slot):
            pltpu.make_async_copy(h_hbm.at[idxv.at[slot, 0]], rowsv.at[slot],
                                  semg).wait()

        def fire_scat(slot):
            pltpu.async_copy(rowsv.at[slot], acc.at[idxv.at[slot, 1]], sems,
                             add=True)

        def wait_scat(slot):
            pltpu.make_async_copy(rowsv.at[slot], acc.at[idxv.at[slot, 1]],
                                  sems).wait()

        fire_idx(0, 0)
        fire_idx(1, 1)
        wait_idx(0)
        fire_gather(0)

        @pl.loop(0, outer // R)
        def _(g):
            for r in range(R):
                t = g * R + r

                @pl.when(t >= 2)
                def _():
                    wait_scat((r + 2) % R)

                @pl.when(t < outer - 2)
                def _():
                    fire_idx(t + 2, (r + 2) % R)

                wait_gather(r)
                fire_scat(r)

                @pl.when(t < outer - 1)
                def _():
                    wait_idx((r + 1) % R)
                    fire_gather((r + 1) % R)

        wait_scat((outer - 2) % R)
        wait_scat((outer - 1) % R)

        plsc.subcore_barrier()

        @pl.loop(0, rps // zrows)
        def _(i):
            pltpu.sync_copy(acc.at[pl.ds(s * rps + i * zrows, zrows), :], zb)
            pltpu.sync_copy(zb,
                            out_hbm.at[c, pl.ds(s * rps + i * zrows, zrows), :])

    return k(ht, ei4)


def _sc_edge_pass_sp(ht, ei4):
    n_pad = ht.shape[0]
    d8 = ht.shape[1]
    nw, outer, _, ch = ei4.shape
    rps = n_pad // NS
    zrows = rps // 17
    R = 4

    @functools.partial(
        pl.kernel,
        out_type=jax.ShapeDtypeStruct((NC, n_pad, d8), jnp.float32),
        mesh=_mesh(),
        compiler_params=_SC_PARAMS,
        scratch_types=[
            pltpu.VMEM((R, 2, ch), jnp.int32),
            pltpu.VMEM((R, ch, d8), jnp.float32),
            pltpu.VMEM((zrows, d8), jnp.float32),
            pltpu.VMEM_SHARED((n_pad, d8), jnp.float32),
            pltpu.VMEM_SHARED((n_pad, d8), jnp.float32),
            pltpu.SemaphoreType.DMA,
            pltpu.SemaphoreType.DMA,
            pltpu.SemaphoreType.DMA,
        ],
    )
    def k(h_hbm, ei_hbm, out_hbm, idxv, rowsv, zb, tbl, acc, semi, semg, sems):
        c = lax.axis_index("c")
        s = lax.axis_index("s")
        w = c * NS + s

        pltpu.sync_copy(h_hbm.at[pl.ds(s * rps, rps), :],
                        tbl.at[pl.ds(s * rps, rps), :])

        @pl.loop(0, zrows)
        def _(i):
            zb[i, :] = jnp.zeros((d8,), jnp.float32)

        @pl.loop(0, rps // zrows)
        def _(i):
            pltpu.sync_copy(zb, acc.at[pl.ds(s * rps + i * zrows, zrows), :])

        plsc.subcore_barrier()

        def fire_idx(t, slot):
            pltpu.async_copy(ei_hbm.at[w, t], idxv.at[slot], semi)

        def wait_idx(slot):
            pltpu.make_async_copy(ei_hbm.at[w, 0], idxv.at[slot], semi).wait()

        def fire_gather(slot):
            pltpu.async_copy(tbl.at[idxv.at[slot, 0]], rowsv.at[slot], semg)

        def wait_gather(slot):
            pltpu.make_async_copy(tbl.at[idxv.at[slot, 0]], rowsv.at[slot],
                                  semg).wait()

        def fire_scat(slot):
            pltpu.async_copy(rowsv.at[slot], acc.at[idxv.at[slot, 1]], sems,
                             add=True)

        def wait_scat(slot):
            pltpu.make_async_copy(rowsv.at[slot], acc.at[idxv.at[slot, 1]],
                                  sems).wait()

        fire_idx(0, 0)
        fire_idx(1, 1)
        wait_idx(0)
        fire_gather(0)

        @pl.loop(0, outer // R)
        def _(g):
            for r in range(R):
                t = g * R + r

                @pl.when(t >= 2)
                def _():
                    wait_scat((r + 2) % R)

                @pl.when(t < outer - 2)
                def _():
                    fire_idx(t + 2, (r + 2) % R)

                wait_gather(r)
                fire_scat(r)

                @pl.when(t < outer - 1)
                def _():
                    wait_idx((r + 1) % R)
                    fire_gather((r + 1) % R)

        wait_scat((outer - 2) % R)
        wait_scat((outer - 1) % R)

        plsc.subcore_barrier()

        @pl.loop(0, rps // zrows)
        def _(i):
            pltpu.sync_copy(acc.at[pl.ds(s * rps + i * zrows, zrows), :], zb)
            pltpu.sync_copy(zb,
                            out_hbm.at[c, pl.ds(s * rps + i * zrows, zrows), :])

    return k(ht, ei4)


def _tc_matmul(a, w, bias=None):
    n_pad, kdim = a.shape
    f = w.shape[1]
    rb = n_pad // 16

    def body(*refs):
        if bias is None:
            ar, wr, o = refs
            o[...] = jnp.dot(ar[...], wr[...],
                             preferred_element_type=jnp.float32)
        else:
            ar, wr, br, o = refs
            o[...] = jnp.dot(ar[...], wr[...],
                             preferred_element_type=jnp.float32) + br[...]

    in_specs = [
        pl.BlockSpec((rb, kdim), lambda i: (i, 0)),
        pl.BlockSpec((kdim, f), lambda i: (0, 0)),
    ]
    args = [a, w]
    if bias is not None:
        in_specs.append(pl.BlockSpec((1, f), lambda i: (0, 0)))
        args.append(bias)
    return pl.pallas_call(
        body,
        grid=(16,),
        in_specs=in_specs,
        out_specs=pl.BlockSpec((rb, f), lambda i: (i, 0)),
        out_shape=jax.ShapeDtypeStruct((n_pad, f), jnp.float32),
    )(*args)


def _tc_scale(deg_t, h1raw):
    n_pad = h1raw.shape[0]
    rb = n_pad // 16
    half = D // 2

    def body(dref, href, dout, haout, hbout):
        dsum = dref[:, 0:1] + dref[:, 1:2] + 1.0
        dcol = lax.rsqrt(dsum)
        de = jnp.broadcast_to(dcol, (rb, D))
        dout[...] = de
        ht = href[...] * de
        haout[...] = ht[:, :half]
        hbout[...] = ht[:, half:]

    return pl.pallas_call(
        body,
        grid=(16,),
        in_specs=[
            pl.BlockSpec((rb, 2), lambda i: (i, 0)),
            pl.BlockSpec((rb, D), lambda i: (i, 0)),
        ],
        out_specs=[
            pl.BlockSpec((rb, D), lambda i: (i, 0)),
            pl.BlockSpec((rb, half), lambda i: (i, 0)),
            pl.BlockSpec((rb, half), lambda i: (i, 0)),
        ],
        out_shape=[
            jax.ShapeDtypeStruct((n_pad, D), jnp.float32),
            jax.ShapeDtypeStruct((n_pad, half), jnp.float32),
            jax.ShapeDtypeStruct((n_pad, half), jnp.float32),
        ],
    )(deg_t, h1raw)


def _tc_combine_split(pa, pb, ha, hb, dinv_e, be):
    n_pad, half = ha.shape
    rb = n_pad // 16

    def body(paref, pbref, haref, hbref, dref, bref, tdout):
        da = dref[:, :half]
        za = (paref[0] + paref[1] + haref[...]) * da + bref[:, :half]
        zb = (pbref[0] + pbref[1] + hbref[...]) * da + bref[:, half:]
        t = jnp.concatenate([jnp.tanh(za), jnp.tanh(zb)], axis=1)
        tdout[...] = t * dref[...]

    return pl.pallas_call(
        body,
        grid=(16,),
        in_specs=[
            pl.BlockSpec((2, rb, half), lambda i: (0, i, 0)),
            pl.BlockSpec((2, rb, half), lambda i: (0, i, 0)),
            pl.BlockSpec((rb, half), lambda i: (i, 0)),
            pl.BlockSpec((rb, half), lambda i: (i, 0)),
            pl.BlockSpec((rb, D), lambda i: (i, 0)),
            pl.BlockSpec((1, D), lambda i: (0, 0)),
        ],
        out_specs=pl.BlockSpec((rb, D), lambda i: (i, 0)),
        out_shape=jax.ShapeDtypeStruct((n_pad, D), jnp.float32),
    )(pa, pb, ha, hb, dinv_e, be)


def _tc_combine(p, htf, dinv_e, be):
    n_pad, dd = htf.shape
    rb = n_pad // 16

    def body(pref, href, dref, bref, tout, tdout):
        de = dref[:, :dd]
        z = (pref[0] + pref[1] + href[...]) * de + bref[...]
        t = jnp.tanh(z)
        tout[...] = t
        tdout[...] = t * de

    return pl.pallas_call(
        body,
        grid=(16,),
        in_specs=[
            pl.BlockSpec((2, rb, dd), lambda i: (0, i, 0)),
            pl.BlockSpec((rb, dd), lambda i: (i, 0)),
            pl.BlockSpec((rb, D), lambda i: (i, 0)),
            pl.BlockSpec((1, dd), lambda i: (0, 0)),
        ],
        out_specs=[
            pl.BlockSpec((rb, dd), lambda i: (i, 0)),
            pl.BlockSpec((rb, dd), lambda i: (i, 0)),
        ],
        out_shape=[
            jax.ShapeDtypeStruct((n_pad, dd), jnp.float32),
            jax.ShapeDtypeStruct((n_pad, dd), jnp.float32),
        ],
    )(p, htf, dinv_e, be)


def _tc_final(t3f, wc, bc, n, hw):
    n_pad, dd = t3f.shape
    fo = wc.shape[1]
    grid = 10
    rb = n // grid

    def body(tref, wref, bref, oout, hout):
        t = tref[...]
        oout[...] = jnp.dot(t, wref[...],
                            preferred_element_type=jnp.float32) + bref[...]
        hout[...] = t[:, :hw]

    return pl.pallas_call(
        body,
        grid=(grid,),
        in_specs=[
            pl.BlockSpec((rb, dd), lambda i: (i, 0)),
            pl.BlockSpec((dd, fo), lambda i: (0, 0)),
            pl.BlockSpec((1, fo), lambda i: (0, 0)),
        ],
        out_specs=[
            pl.BlockSpec((rb, fo), lambda i: (i, 0)),
            pl.BlockSpec((rb, hw), lambda i: (i, 0)),
        ],
        out_shape=[
            jax.ShapeDtypeStruct((n, fo), jnp.float32),
            jax.ShapeDtypeStruct((n, hw), jnp.float32),
        ],
    )(t3f, wc, bc)


def _pad_w(w, width=D):
    k, f = w.shape
    return jnp.zeros((k, width), jnp.float32).at[:, :f].set(w)


def _pad_b(b, width=D):
    return jnp.zeros((1, width), jnp.float32).at[0, : b.shape[0]].set(b)


def kernel(x, edge_index, W1, b1, W2, b2, W3, b3, Wc, bc):
    n = x.shape[0]
    e = edge_index.shape[1]
    n_pad = -(-(n + 1) // 128) * 128
    m = n_pad * D // 128

    src = edge_index[0].astype(jnp.int32)
    dst = edge_index[1].astype(jnp.int32)

    rows = -(-e // LANES)
    rows_w = -(-rows // (NC * NS))
    rows_w = -(-rows_w // SB) * SB
    rows_p = rows_w * NC * NS
    pad = rows_p * LANES - e
    scratch_rows = n_pad - n
    src_p = jnp.concatenate([src, jnp.zeros((pad,), jnp.int32)])
    dst_p = jnp.concatenate(
        [dst, n + (jnp.arange(pad, dtype=jnp.int32) % scratch_rows)])

    nw = NC * NS
    ew = rows_w * LANES
    ch = 320
    outer = ew // ch
    ei4 = jnp.stack([src_p.reshape(nw, outer, ch),
                     dst_p.reshape(nw, outer, ch)], axis=2)

    d8 = 8
    xp = jnp.pad(x, ((0, n_pad - n), (0, 0)))
    w1p = _pad_w(W1)
    w2p, w3p = _pad_w(W2, d8), _pad_w(W3, d8)
    wcp = jnp.zeros((d8, Wc.shape[1]), jnp.float32).at[: Wc.shape[0]].set(Wc)
    be1 = _pad_b(b1)
    be2, be3 = _pad_b(b2, d8), _pad_b(b3, d8)
    bcp = bc[None, :]

    h1raw = _tc_matmul(xp, w1p)
    degp = _sc_degree(dst_p, n_pad)

    deg_t = jnp.transpose(degp)
    dinv_e, h1ta, h1tb = _tc_scale(deg_t, h1raw)

    pa = _sc_edge_pass_sp(h1ta, ei4)
    pb = _sc_edge_pass_sp(h1tb, ei4)
    td1 = _tc_combine_split(pa, pb, h1ta, h1tb, dinv_e, be1)
    ht2 = _tc_matmul(td1, w2p)

    p = _sc_edge_pass_sp(ht2, ei4)
    _, td2 = _tc_combine(p, ht2, dinv_e, be2)
    ht3 = _tc_matmul(td2, w3p)

    p = _sc_edge_pass_sp(ht3, ei4)
    t3, _ = _tc_combine(p, ht3, dinv_e, be3)

    out, h = _tc_final(t3, wcp, bcp, n, W3.shape[1])
    return (out, h)
